# initial kernel scaffold (unmeasured)
import jax
import jax.numpy as jnp
from jax import lax
from jax.experimental import pallas as pl
from jax.experimental.pallas import tpu as pltpu

N_DEV = 4
M = 4096
N = 2048
CHUNK = M // N_DEV


def kernel(x, w_mat, scale_x, scale_w):
    def body(x_ref, w_ref, sx_ref, sw_ref, out_ref, comm_ref, send_sems, recv_sems):
        me = lax.axis_index("i")
        left = lax.rem(me + N_DEV - 1, N_DEV)
        right = lax.rem(me + 1, N_DEV)

        barrier_sem = pltpu.get_barrier_semaphore()
        for nbr in [left, right]:
            pl.semaphore_signal(
                barrier_sem, inc=1,
                device_id=(nbr,), device_id_type=pl.DeviceIdType.MESH,
            )
        pl.semaphore_wait(barrier_sem, 2)

        scale = sx_ref[0] * sw_ref[0]
        acc = jnp.dot(x_ref[...], w_ref[...], preferred_element_type=jnp.float32)
        out_ref[...] = acc * scale

        for h in range(N_DEV - 1):
            s = h % 2
            r = (h + 1) % 2
            send_chunk = lax.rem(me + N_DEV - h, N_DEV)
            recv_chunk = lax.rem(me + N_DEV - h - 1, N_DEV)
            comm_ref[s, :, :] = out_ref[pl.ds(send_chunk * CHUNK, CHUNK), :]
            rdma = pltpu.make_async_remote_copy(
                src_ref=comm_ref.at[s],
                dst_ref=comm_ref.at[r],
                send_sem=send_sems.at[s],
                recv_sem=recv_sems.at[r],
                device_id=(right,),
                device_id_type=pl.DeviceIdType.MESH,
            )
            rdma.start()
            rdma.wait()
            out_ref[pl.ds(recv_chunk * CHUNK, CHUNK), :] = (
                out_ref[pl.ds(recv_chunk * CHUNK, CHUNK), :] + comm_ref[r, :, :]
            )

        for g in range(N_DEV - 1):
            h = (N_DEV - 1) + g
            s = h % 2
            r = (h + 1) % 2
            send_chunk = lax.rem(me + 1 + N_DEV - g, N_DEV)
            recv_chunk = lax.rem(me + N_DEV - g, N_DEV)
            comm_ref[s, :, :] = out_ref[pl.ds(send_chunk * CHUNK, CHUNK), :]
            rdma = pltpu.make_async_remote_copy(
                src_ref=comm_ref.at[s],
                dst_ref=comm_ref.at[r],
                send_sem=send_sems.at[s],
                recv_sem=recv_sems.at[r],
                device_id=(right,),
                device_id_type=pl.DeviceIdType.MESH,
            )
            rdma.start()
            rdma.wait()
            out_ref[pl.ds(recv_chunk * CHUNK, CHUNK), :] = comm_ref[r, :, :]

    return pl.pallas_call(
        body,
        out_shape=jax.ShapeDtypeStruct((M, N), jnp.float32),
        in_specs=[
            pl.BlockSpec(memory_space=pltpu.VMEM),
            pl.BlockSpec(memory_space=pltpu.VMEM),
            pl.BlockSpec(memory_space=pltpu.SMEM),
            pl.BlockSpec(memory_space=pltpu.SMEM),
        ],
        out_specs=pl.BlockSpec(memory_space=pltpu.VMEM),
        scratch_shapes=[
            pltpu.VMEM((2, CHUNK, N), jnp.float32),
            pltpu.SemaphoreType.DMA((2,)),
            pltpu.SemaphoreType.DMA((2,)),
        ],
        compiler_params=pltpu.CompilerParams(collective_id=0),
    )(x, w_mat, scale_x, scale_w)


# baseline (device time: 616375 ns/iter reference)
import jax
import jax.numpy as jnp
from jax import lax
from jax.experimental import pallas as pl
from jax.experimental.pallas import tpu as pltpu

N_DEV = 4
M = 4096
N = 2048
CHUNK = M // N_DEV


def kernel(x, w_mat, scale_x, scale_w):
    def body(x_ref, w_ref, sx_ref, sw_ref, out_ref,
             comm_ref, send_sems, recv_sems, copy_sem):
        me = lax.axis_index("i")
        left = lax.rem(me + N_DEV - 1, N_DEV)
        right = lax.rem(me + 1, N_DEV)

        barrier_sem = pltpu.get_barrier_semaphore()
        for nbr in [left, right]:
            pl.semaphore_signal(
                barrier_sem, inc=1,
                device_id=(nbr,), device_id_type=pl.DeviceIdType.MESH,
            )
        pl.semaphore_wait(barrier_sem, 2)

        scale = sx_ref[0] * sw_ref[0]

        def partial_chunk(c):
            return jnp.dot(
                x_ref[pl.ds(c * CHUNK, CHUNK), :], w_ref[...],
                preferred_element_type=jnp.float32,
            ) * scale

        def ring_send(s, r):
            rdma = pltpu.make_async_remote_copy(
                src_ref=comm_ref.at[s],
                dst_ref=comm_ref.at[r],
                send_sem=send_sems.at[s],
                recv_sem=recv_sems.at[r],
                device_id=(right,),
                device_id_type=pl.DeviceIdType.MESH,
            )
            rdma.start()
            rdma.wait()

        def store_out(c, slot):
            cp = pltpu.make_async_copy(
                comm_ref.at[slot],
                out_ref.at[pl.ds(c * CHUNK, CHUNK), :],
                copy_sem,
            )
            cp.start()
            cp.wait()

        for h in range(N_DEV - 1):
            s = h % 2
            r = (h + 1) % 2
            c_send = lax.rem(me + N_DEV - h, N_DEV)
            val = partial_chunk(c_send)
            if h > 0:
                val = val + comm_ref[s, :, :]
            comm_ref[s, :, :] = val
            ring_send(s, r)

        c_mine = lax.rem(me + 1, N_DEV)
        comm_ref[1, :, :] = partial_chunk(c_mine) + comm_ref[1, :, :]
        store_out(c_mine, 1)

        for g in range(N_DEV - 1):
            h = (N_DEV - 1) + g
            s = h % 2
            r = (h + 1) % 2
            ring_send(s, r)
            c_recv = lax.rem(me + N_DEV - g, N_DEV)
            store_out(c_recv, r)

    return pl.pallas_call(
        body,
        out_shape=jax.ShapeDtypeStruct((M, N), jnp.float32),
        in_specs=[
            pl.BlockSpec(memory_space=pltpu.MemorySpace.VMEM),
            pl.BlockSpec(memory_space=pltpu.MemorySpace.VMEM),
            pl.BlockSpec(memory_space=pltpu.MemorySpace.SMEM),
            pl.BlockSpec(memory_space=pltpu.MemorySpace.SMEM),
        ],
        out_specs=pl.BlockSpec(memory_space=pl.ANY),
        scratch_shapes=[
            pltpu.VMEM((2, CHUNK, N), jnp.float32),
            pltpu.SemaphoreType.DMA((2,)),
            pltpu.SemaphoreType.DMA((2,)),
            pltpu.SemaphoreType.DMA,
        ],
        compiler_params=pltpu.CompilerParams(collective_id=0),
    )(x, w_mat, scale_x, scale_w)


# device time: 192065 ns/iter; 3.2092x vs baseline; 3.2092x over previous
import jax
import jax.numpy as jnp
from jax import lax
from jax.experimental import pallas as pl
from jax.experimental.pallas import tpu as pltpu

N_DEV = 4
M = 4096
N = 2048
CHUNK = M // N_DEV
HALF = CHUNK // 2


def kernel(x, w_mat, scale_x, scale_w):
    def body(x_ref, w_ref, sx_ref, sw_ref, out_ref,
             w16_ref, comm_cw, comm_ccw, stage_ref,
             send_cw, recv_cw, send_ccw, recv_ccw, stage_sems):
        me = lax.axis_index("i")
        left = lax.rem(me + N_DEV - 1, N_DEV)
        right = lax.rem(me + 1, N_DEV)

        barrier_sem = pltpu.get_barrier_semaphore()
        for nbr in [left, right]:
            pl.semaphore_signal(
                barrier_sem, inc=1,
                device_id=(nbr,), device_id_type=pl.DeviceIdType.MESH,
            )
        pl.semaphore_wait(barrier_sem, 2)

        scale = sx_ref[0] * sw_ref[0]
        w16_ref[...] = w_ref[...].astype(jnp.bfloat16)

        def partial_half(c, off):
            xh = x_ref[pl.ds(c * CHUNK + off, HALF), :].astype(jnp.bfloat16)
            return jnp.dot(
                xh, w16_ref[...], preferred_element_type=jnp.float32
            ) * scale

        def rdma(buf, sems_s, sems_r, s, r, dst):
            return pltpu.make_async_remote_copy(
                src_ref=buf.at[s],
                dst_ref=buf.at[r],
                send_sem=sems_s.at[s],
                recv_sem=sems_r.at[r],
                device_id=(dst,),
                device_id_type=pl.DeviceIdType.MESH,
            )

        stage_started = [False, False]

        def store_half(dir_slot, val_f32, c, off):
            if stage_started[dir_slot]:
                pltpu.make_async_copy(
                    stage_ref.at[dir_slot],
                    stage_ref.at[dir_slot],
                    stage_sems.at[dir_slot],
                ).wait()
            stage_ref[dir_slot, :, :] = val_f32
            cp = pltpu.make_async_copy(
                stage_ref.at[dir_slot],
                out_ref.at[pl.ds(c * CHUNK + off, HALF), :],
                stage_sems.at[dir_slot],
            )
            cp.start()
            stage_started[dir_slot] = True

        p_cw = partial_half(me, 0)
        p_ccw = partial_half(me, HALF)
        for h in range(N_DEV - 1):
            s = h % 2
            r = (h + 1) % 2
            if h > 0:
                rdma(comm_cw, send_cw, recv_cw, s, s, right).wait_recv()
                rdma(comm_ccw, send_ccw, recv_ccw, s, s, left).wait_recv()
                p_cw = p_cw + comm_cw[s, :, :].astype(jnp.float32)
                p_ccw = p_ccw + comm_ccw[s, :, :].astype(jnp.float32)
            if h >= 2:
                rdma(comm_cw, send_cw, recv_cw, s, s, right).wait_send()
                rdma(comm_ccw, send_ccw, recv_ccw, s, s, left).wait_send()
            comm_cw[s, :, :] = p_cw.astype(jnp.bfloat16)
            comm_ccw[s, :, :] = p_ccw.astype(jnp.bfloat16)
            rdma(comm_cw, send_cw, recv_cw, s, r, right).start()
            rdma(comm_ccw, send_ccw, recv_ccw, s, r, left).start()
            c_cw = lax.rem(me + N_DEV - h - 1, N_DEV)
            c_ccw = lax.rem(me + h + 1, N_DEV)
            p_cw = partial_half(c_cw, 0)
            p_ccw = partial_half(c_ccw, HALF)

        rdma(comm_cw, send_cw, recv_cw, 1, 1, right).wait_recv()
        rdma(comm_ccw, send_ccw, recv_ccw, 1, 1, left).wait_recv()
        v_cw = p_cw + comm_cw[1, :, :].astype(jnp.float32)
        v_ccw = p_ccw + comm_ccw[1, :, :].astype(jnp.float32)
        rdma(comm_cw, send_cw, recv_cw, 1, 1, right).wait_send()
        rdma(comm_ccw, send_ccw, recv_ccw, 1, 1, left).wait_send()
        comm_cw[1, :, :] = v_cw.astype(jnp.bfloat16)
        comm_ccw[1, :, :] = v_ccw.astype(jnp.bfloat16)

        c_cw_fin = lax.rem(me + 1, N_DEV)
        c_ccw_fin = lax.rem(me + 3, N_DEV)

        for g in range(N_DEV - 1):
            h = (N_DEV - 1) + g
            s = h % 2
            r = (h + 1) % 2
            if g > 0:
                rdma(comm_cw, send_cw, recv_cw, s, s, right).wait_recv()
                rdma(comm_ccw, send_ccw, recv_ccw, s, s, left).wait_recv()
                rdma(comm_cw, send_cw, recv_cw, s, s, right).wait_send()
                rdma(comm_ccw, send_ccw, recv_ccw, s, s, left).wait_send()
            rdma(comm_cw, send_cw, recv_cw, s, r, right).start()
            rdma(comm_ccw, send_ccw, recv_ccw, s, r, left).start()
            if g == 0:
                store_half(0, v_cw, c_cw_fin, 0)
                store_half(1, v_ccw, c_ccw_fin, HALF)
            else:
                c_cw = lax.rem(me + 1 + N_DEV - g, N_DEV)
                c_ccw = lax.rem(me + 3 + g, N_DEV)
                store_half(0, comm_cw[s, :, :].astype(jnp.float32), c_cw, 0)
                store_half(1, comm_ccw[s, :, :].astype(jnp.float32), c_ccw, HALF)

        rdma(comm_cw, send_cw, recv_cw, 0, 0, right).wait_recv()
        rdma(comm_ccw, send_ccw, recv_ccw, 0, 0, left).wait_recv()
        c_cw = lax.rem(me + 2, N_DEV)
        c_ccw = lax.rem(me + 2, N_DEV)
        store_half(0, comm_cw[0, :, :].astype(jnp.float32), c_cw, 0)
        store_half(1, comm_ccw[0, :, :].astype(jnp.float32), c_ccw, HALF)

        rdma(comm_cw, send_cw, recv_cw, 0, 0, right).wait_send()
        rdma(comm_ccw, send_ccw, recv_ccw, 0, 0, left).wait_send()
        rdma(comm_cw, send_cw, recv_cw, 1, 1, right).wait_send()
        rdma(comm_ccw, send_ccw, recv_ccw, 1, 1, left).wait_send()
        for d in range(2):
            pltpu.make_async_copy(
                stage_ref.at[d], stage_ref.at[d], stage_sems.at[d]
            ).wait()

    return pl.pallas_call(
        body,
        out_shape=jax.ShapeDtypeStruct((M, N), jnp.float32),
        in_specs=[
            pl.BlockSpec(memory_space=pltpu.MemorySpace.VMEM),
            pl.BlockSpec(memory_space=pltpu.MemorySpace.VMEM),
            pl.BlockSpec(memory_space=pltpu.MemorySpace.SMEM),
            pl.BlockSpec(memory_space=pltpu.MemorySpace.SMEM),
        ],
        out_specs=pl.BlockSpec(memory_space=pl.ANY),
        scratch_shapes=[
            pltpu.VMEM((CHUNK, N), jnp.bfloat16),
            pltpu.VMEM((2, HALF, N), jnp.bfloat16),
            pltpu.VMEM((2, HALF, N), jnp.bfloat16),
            pltpu.VMEM((2, HALF, N), jnp.float32),
            pltpu.SemaphoreType.DMA((2,)),
            pltpu.SemaphoreType.DMA((2,)),
            pltpu.SemaphoreType.DMA((2,)),
            pltpu.SemaphoreType.DMA((2,)),
            pltpu.SemaphoreType.DMA((2,)),
        ],
        compiler_params=pltpu.CompilerParams(
            collective_id=0,
            vmem_limit_bytes=100 * 1024 * 1024,
        ),
    )(x, w_mat, scale_x, scale_w)


# device time: 179769 ns/iter; 3.4287x vs baseline; 1.0684x over previous
import jax
import jax.numpy as jnp
from jax import lax
from jax.experimental import pallas as pl
from jax.experimental.pallas import tpu as pltpu

N_DEV = 4
M = 4096
N = 2048
CHUNK = M // N_DEV
HALF = CHUNK // 2
NSUB = 2
SUB = HALF // NSUB


def kernel(x, w_mat, scale_x, scale_w):
    def body(x_ref, w_ref, sx_ref, sw_ref, out_ref,
             w16_ref, comm_cw, comm_ccw, stage_ref,
             send_cw, recv_cw, send_ccw, recv_ccw, stage_sems):
        me = lax.axis_index("i")
        left = lax.rem(me + N_DEV - 1, N_DEV)
        right = lax.rem(me + 1, N_DEV)

        scale = sx_ref[0] * sw_ref[0]
        w16_ref[...] = w_ref[...].astype(jnp.bfloat16)

        def partial_sub(c, off, j):
            xh = x_ref[pl.ds(c * CHUNK + off + j * SUB, SUB), :].astype(
                jnp.bfloat16)
            return jnp.dot(
                xh, w16_ref[...], preferred_element_type=jnp.float32
            ) * scale

        def rdma(buf, sems_s, sems_r, s, r, j, dst):
            return pltpu.make_async_remote_copy(
                src_ref=buf.at[s, pl.ds(j * SUB, SUB), :],
                dst_ref=buf.at[r, pl.ds(j * SUB, SUB), :],
                send_sem=sems_s.at[s, j],
                recv_sem=sems_r.at[r, j],
                device_id=(dst,),
                device_id_type=pl.DeviceIdType.MESH,
            )

        def cw(s, r, j):
            return rdma(comm_cw, send_cw, recv_cw, s, r, j, right)

        def ccw(s, r, j):
            return rdma(comm_ccw, send_ccw, recv_ccw, s, r, j, left)

        stage_started = [False, False]

        def store_half(dir_slot, val_f32, c, off):
            if stage_started[dir_slot]:
                pltpu.make_async_copy(
                    stage_ref.at[dir_slot], stage_ref.at[dir_slot],
                    stage_sems.at[dir_slot],
                ).wait()
            stage_ref[dir_slot, :, :] = val_f32
            pltpu.make_async_copy(
                stage_ref.at[dir_slot],
                out_ref.at[pl.ds(c * CHUNK + off, HALF), :],
                stage_sems.at[dir_slot],
            ).start()
            stage_started[dir_slot] = True

        p_cw = [partial_sub(me, 0, j) for j in range(NSUB)]
        p_ccw = [partial_sub(me, HALF, j) for j in range(NSUB)]

        barrier_sem = pltpu.get_barrier_semaphore()
        for nbr in [left, right]:
            pl.semaphore_signal(
                barrier_sem, inc=1,
                device_id=(nbr,), device_id_type=pl.DeviceIdType.MESH,
            )
        pl.semaphore_wait(barrier_sem, 2)

        for h in range(N_DEV - 1):
            s = h % 2
            r = (h + 1) % 2
            for j in range(NSUB):
                if h > 0:
                    cw(s, s, j).wait_recv()
                    ccw(s, s, j).wait_recv()
                    p_cw[j] = p_cw[j] + comm_cw[
                        s, pl.ds(j * SUB, SUB), :].astype(jnp.float32)
                    p_ccw[j] = p_ccw[j] + comm_ccw[
                        s, pl.ds(j * SUB, SUB), :].astype(jnp.float32)
                if h >= 2:
                    cw(s, s, j).wait_send()
                    ccw(s, s, j).wait_send()
                comm_cw[s, pl.ds(j * SUB, SUB), :] = p_cw[j].astype(jnp.bfloat16)
                comm_ccw[s, pl.ds(j * SUB, SUB), :] = p_ccw[j].astype(jnp.bfloat16)
                cw(s, r, j).start()
                ccw(s, r, j).start()
            c_cw = lax.rem(me + N_DEV - h - 1, N_DEV)
            c_ccw = lax.rem(me + h + 1, N_DEV)
            p_cw = [partial_sub(c_cw, 0, j) for j in range(NSUB)]
            p_ccw = [partial_sub(c_ccw, HALF, j) for j in range(NSUB)]

        v_cw, v_ccw = [None, None], [None, None]
        for j in range(NSUB):
            cw(1, 1, j).wait_recv()
            ccw(1, 1, j).wait_recv()
            v_cw[j] = p_cw[j] + comm_cw[
                1, pl.ds(j * SUB, SUB), :].astype(jnp.float32)
            v_ccw[j] = p_ccw[j] + comm_ccw[
                1, pl.ds(j * SUB, SUB), :].astype(jnp.float32)
            cw(1, 1, j).wait_send()
            ccw(1, 1, j).wait_send()
            comm_cw[1, pl.ds(j * SUB, SUB), :] = v_cw[j].astype(jnp.bfloat16)
            comm_ccw[1, pl.ds(j * SUB, SUB), :] = v_ccw[j].astype(jnp.bfloat16)
            cw(1, 0, j).start()
            ccw(1, 0, j).start()

        c_cw_fin = lax.rem(me + 1, N_DEV)
        c_ccw_fin = lax.rem(me + 3, N_DEV)
        store_half(0, jnp.concatenate(v_cw, axis=0), c_cw_fin, 0)
        store_half(1, jnp.concatenate(v_ccw, axis=0), c_ccw_fin, HALF)

        for g in range(1, N_DEV - 1):
            h = (N_DEV - 1) + g
            s = h % 2
            r = (h + 1) % 2
            for j in range(NSUB):
                cw(s, s, j).wait_recv()
                ccw(s, s, j).wait_recv()
                cw(s, s, j).wait_send()
                ccw(s, s, j).wait_send()
                cw(s, r, j).start()
                ccw(s, r, j).start()
            c_cw = lax.rem(me + 1 + N_DEV - g, N_DEV)
            c_ccw = lax.rem(me + 3 + g, N_DEV)
            store_half(0, comm_cw[s, :, :].astype(jnp.float32), c_cw, 0)
            store_half(1, comm_ccw[s, :, :].astype(jnp.float32), c_ccw, HALF)

        for j in range(NSUB):
            cw(0, 0, j).wait_recv()
            ccw(0, 0, j).wait_recv()
        c_last = lax.rem(me + 2, N_DEV)
        store_half(0, comm_cw[0, :, :].astype(jnp.float32), c_last, 0)
        store_half(1, comm_ccw[0, :, :].astype(jnp.float32), c_last, HALF)

        for j in range(NSUB):
            cw(0, 0, j).wait_send()
            ccw(0, 0, j).wait_send()
            cw(1, 1, j).wait_send()
            ccw(1, 1, j).wait_send()
        for d in range(2):
            pltpu.make_async_copy(
                stage_ref.at[d], stage_ref.at[d], stage_sems.at[d]
            ).wait()

    return pl.pallas_call(
        body,
        out_shape=jax.ShapeDtypeStruct((M, N), jnp.float32),
        in_specs=[
            pl.BlockSpec(memory_space=pltpu.MemorySpace.VMEM),
            pl.BlockSpec(memory_space=pltpu.MemorySpace.VMEM),
            pl.BlockSpec(memory_space=pltpu.MemorySpace.SMEM),
            pl.BlockSpec(memory_space=pltpu.MemorySpace.SMEM),
        ],
        out_specs=pl.BlockSpec(memory_space=pl.ANY),
        scratch_shapes=[
            pltpu.VMEM((CHUNK, N), jnp.bfloat16),
            pltpu.VMEM((2, HALF, N), jnp.bfloat16),
            pltpu.VMEM((2, HALF, N), jnp.bfloat16),
            pltpu.VMEM((2, HALF, N), jnp.float32),
            pltpu.SemaphoreType.DMA((2, NSUB)),
            pltpu.SemaphoreType.DMA((2, NSUB)),
            pltpu.SemaphoreType.DMA((2, NSUB)),
            pltpu.SemaphoreType.DMA((2, NSUB)),
            pltpu.SemaphoreType.DMA((2,)),
        ],
        compiler_params=pltpu.CompilerParams(
            collective_id=0,
            vmem_limit_bytes=100 * 1024 * 1024,
        ),
    )(x, w_mat, scale_x, scale_w)


# device time: 179584 ns/iter; 3.4322x vs baseline; 1.0010x over previous
import jax
import jax.numpy as jnp
from jax import lax
from jax.experimental import pallas as pl
from jax.experimental.pallas import tpu as pltpu

N_DEV = 4
M = 4096
N = 2048
CHUNK = M // N_DEV
HALF = CHUNK // 2
NSUB = 2
SUB = HALF // NSUB


def kernel(x, w_mat, scale_x, scale_w):
    def body(x_ref, w_ref, sx_ref, sw_ref, out_ref,
             w16_ref, comm_cw, comm_ccw, stage_ref,
             send_cw, recv_cw, send_ccw, recv_ccw, stage_sems):
        me = lax.axis_index("i")
        left = lax.rem(me + N_DEV - 1, N_DEV)
        right = lax.rem(me + 1, N_DEV)

        scale = sx_ref[0] * sw_ref[0]
        w16_ref[...] = w_ref[...].astype(jnp.bfloat16)

        def partial_sub(c, off, j):
            xh = x_ref[pl.ds(c * CHUNK + off + j * SUB, SUB), :].astype(
                jnp.bfloat16)
            return jnp.dot(
                xh, w16_ref[...], preferred_element_type=jnp.float32
            ) * scale

        def rdma(buf, sems_s, sems_r, s, r, j, dst):
            return pltpu.make_async_remote_copy(
                src_ref=buf.at[s, pl.ds(j * SUB, SUB), :],
                dst_ref=buf.at[r, pl.ds(j * SUB, SUB), :],
                send_sem=sems_s.at[s, j],
                recv_sem=sems_r.at[r, j],
                device_id=(dst,),
                device_id_type=pl.DeviceIdType.MESH,
            )

        def cw(s, r, j):
            return rdma(comm_cw, send_cw, recv_cw, s, r, j, right)

        def ccw(s, r, j):
            return rdma(comm_ccw, send_ccw, recv_ccw, s, r, j, left)

        stage_started = [False, False]

        def store_half(dir_slot, val_f32, c, off):
            if stage_started[dir_slot]:
                pltpu.make_async_copy(
                    stage_ref.at[dir_slot], stage_ref.at[dir_slot],
                    stage_sems.at[dir_slot],
                ).wait()
            stage_ref[dir_slot, :, :] = val_f32
            pltpu.make_async_copy(
                stage_ref.at[dir_slot],
                out_ref.at[pl.ds(c * CHUNK + off, HALF), :],
                stage_sems.at[dir_slot],
            ).start()
            stage_started[dir_slot] = True

        p_cw = [partial_sub(me, 0, j) for j in range(NSUB)]
        p_ccw = [partial_sub(me, HALF, j) for j in range(NSUB)]

        barrier_sem = pltpu.get_barrier_semaphore()
        for nbr in [left, right]:
            pl.semaphore_signal(
                barrier_sem, inc=1,
                device_id=(nbr,), device_id_type=pl.DeviceIdType.MESH,
            )
        pl.semaphore_wait(barrier_sem, 2)

        for h in range(N_DEV - 1):
            s = h % 2
            r = (h + 1) % 2
            for j in range(NSUB):
                if h > 0:
                    cw(s, s, j).wait_recv()
                    ccw(s, s, j).wait_recv()
                    p_cw[j] = p_cw[j] + comm_cw[
                        s, pl.ds(j * SUB, SUB), :].astype(jnp.float32)
                    p_ccw[j] = p_ccw[j] + comm_ccw[
                        s, pl.ds(j * SUB, SUB), :].astype(jnp.float32)
                if h >= 2:
                    cw(s, s, j).wait_send()
                    ccw(s, s, j).wait_send()
                comm_cw[s, pl.ds(j * SUB, SUB), :] = p_cw[j].astype(jnp.bfloat16)
                comm_ccw[s, pl.ds(j * SUB, SUB), :] = p_ccw[j].astype(jnp.bfloat16)
                cw(s, r, j).start()
                ccw(s, r, j).start()
            c_cw = lax.rem(me + N_DEV - h - 1, N_DEV)
            c_ccw = lax.rem(me + h + 1, N_DEV)
            p_cw = [partial_sub(c_cw, 0, j) for j in range(NSUB)]
            p_ccw = [partial_sub(c_ccw, HALF, j) for j in range(NSUB)]

        v_cw, v_ccw = [None, None], [None, None]
        for j in range(NSUB):
            cw(1, 1, j).wait_recv()
            ccw(1, 1, j).wait_recv()
            v_cw[j] = p_cw[j] + comm_cw[
                1, pl.ds(j * SUB, SUB), :].astype(jnp.float32)
            v_ccw[j] = p_ccw[j] + comm_ccw[
                1, pl.ds(j * SUB, SUB), :].astype(jnp.float32)
            cw(1, 1, j).wait_send()
            ccw(1, 1, j).wait_send()
            comm_cw[1, pl.ds(j * SUB, SUB), :] = v_cw[j].astype(jnp.bfloat16)
            comm_ccw[1, pl.ds(j * SUB, SUB), :] = v_ccw[j].astype(jnp.bfloat16)
            cw(1, 0, j).start()
            ccw(1, 0, j).start()

        c_cw_fin = lax.rem(me + 1, N_DEV)
        c_ccw_fin = lax.rem(me + 3, N_DEV)
        store_half(0, jnp.concatenate(v_cw, axis=0), c_cw_fin, 0)
        store_half(1, jnp.concatenate(v_ccw, axis=0), c_ccw_fin, HALF)

        for g in range(1, N_DEV - 1):
            h = (N_DEV - 1) + g
            s = h % 2
            r = (h + 1) % 2
            for j in range(NSUB):
                cw(s, s, j).wait_recv()
                ccw(s, s, j).wait_recv()
                cw(s, s, j).wait_send()
                ccw(s, s, j).wait_send()
                cw(s, r, j).start()
                ccw(s, r, j).start()
            c_cw = lax.rem(me + 1 + N_DEV - g, N_DEV)
            c_ccw = lax.rem(me + 3 + g, N_DEV)
            store_half(0, comm_cw[s, :, :].astype(jnp.float32), c_cw, 0)
            store_half(1, comm_ccw[s, :, :].astype(jnp.float32), c_ccw, HALF)

        for j in range(NSUB):
            cw(0, 0, j).wait_recv()
            ccw(0, 0, j).wait_recv()
        c_last = lax.rem(me + 2, N_DEV)
        store_half(0, comm_cw[0, :, :].astype(jnp.float32), c_last, 0)
        store_half(1, comm_ccw[0, :, :].astype(jnp.float32), c_last, HALF)

        for j in range(NSUB):
            cw(0, 0, j).wait_send()
            ccw(0, 0, j).wait_send()
            cw(1, 1, j).wait_send()
            ccw(1, 1, j).wait_send()
        for d in range(2):
            pltpu.make_async_copy(
                stage_ref.at[d], stage_ref.at[d], stage_sems.at[d]
            ).wait()

    return pl.pallas_call(
        body,
        out_shape=jax.ShapeDtypeStruct((M, N), jnp.float32),
        in_specs=[
            pl.BlockSpec(memory_space=pltpu.MemorySpace.VMEM),
            pl.BlockSpec(memory_space=pltpu.MemorySpace.VMEM),
            pl.BlockSpec(memory_space=pltpu.MemorySpace.SMEM),
            pl.BlockSpec(memory_space=pltpu.MemorySpace.SMEM),
        ],
        out_specs=pl.BlockSpec(memory_space=pl.ANY),
        scratch_shapes=[
            pltpu.VMEM((CHUNK, N), jnp.bfloat16),
            pltpu.VMEM((2, HALF, N), jnp.bfloat16),
            pltpu.VMEM((2, HALF, N), jnp.bfloat16),
            pltpu.VMEM((2, HALF, N), jnp.float32),
            pltpu.SemaphoreType.DMA((2, NSUB)),
            pltpu.SemaphoreType.DMA((2, NSUB)),
            pltpu.SemaphoreType.DMA((2, NSUB)),
            pltpu.SemaphoreType.DMA((2, NSUB)),
            pltpu.SemaphoreType.DMA((2,)),
        ],
        compiler_params=pltpu.CompilerParams(
            collective_id=0,
            vmem_limit_bytes=100 * 1024 * 1024,
            skip_device_barrier=True,
        ),
    )(x, w_mat, scale_x, scale_w)


# device time: 176201 ns/iter; 3.4981x vs baseline; 1.0192x over previous
import jax
import jax.numpy as jnp
from jax import lax
from jax.experimental import pallas as pl
from jax.experimental.pallas import tpu as pltpu

N_DEV = 4
M = 4096
N = 2048
CHUNK = M // N_DEV
HALF = CHUNK // 2
NSUB = 4
SUB = HALF // NSUB


def kernel(x, w_mat, scale_x, scale_w):
    def body(x_ref, w_ref, sx_ref, sw_ref, out_ref,
             w16_ref, comm_cw, comm_ccw, stage_ref,
             send_cw, recv_cw, send_ccw, recv_ccw, stage_sems):
        me = lax.axis_index("i")
        left = lax.rem(me + N_DEV - 1, N_DEV)
        right = lax.rem(me + 1, N_DEV)

        scale = sx_ref[0] * sw_ref[0]
        w16_ref[...] = w_ref[...].astype(jnp.bfloat16)

        def partial_sub(c, off, j):
            xh = x_ref[pl.ds(c * CHUNK + off + j * SUB, SUB), :].astype(
                jnp.bfloat16)
            return jnp.dot(
                xh, w16_ref[...], preferred_element_type=jnp.float32
            ) * scale

        def rdma(buf, sems_s, sems_r, s, r, j, dst):
            return pltpu.make_async_remote_copy(
                src_ref=buf.at[s, pl.ds(j * SUB, SUB), :],
                dst_ref=buf.at[r, pl.ds(j * SUB, SUB), :],
                send_sem=sems_s.at[s, j],
                recv_sem=sems_r.at[r, j],
                device_id=(dst,),
                device_id_type=pl.DeviceIdType.MESH,
            )

        def cw(s, r, j):
            return rdma(comm_cw, send_cw, recv_cw, s, r, j, right)

        def ccw(s, r, j):
            return rdma(comm_ccw, send_ccw, recv_ccw, s, r, j, left)

        stage_started = [False, False]

        def store_half(dir_slot, val_f32, c, off):
            if stage_started[dir_slot]:
                pltpu.make_async_copy(
                    stage_ref.at[dir_slot], stage_ref.at[dir_slot],
                    stage_sems.at[dir_slot],
                ).wait()
            stage_ref[dir_slot, :, :] = val_f32
            pltpu.make_async_copy(
                stage_ref.at[dir_slot],
                out_ref.at[pl.ds(c * CHUNK + off, HALF), :],
                stage_sems.at[dir_slot],
            ).start()
            stage_started[dir_slot] = True

        p_cw = [None] * NSUB
        p_ccw = [None] * NSUB
        p_cw[0] = partial_sub(me, 0, 0)
        p_ccw[0] = partial_sub(me, HALF, 0)

        barrier_sem = pltpu.get_barrier_semaphore()
        for nbr in [left, right]:
            pl.semaphore_signal(
                barrier_sem, inc=1,
                device_id=(nbr,), device_id_type=pl.DeviceIdType.MESH,
            )
        pl.semaphore_wait(barrier_sem, 2)

        for h in range(N_DEV - 1):
            s = h % 2
            r = (h + 1) % 2
            for j in range(NSUB):
                if h == 0 and j > 0:
                    p_cw[j] = partial_sub(me, 0, j)
                    p_ccw[j] = partial_sub(me, HALF, j)
                if h > 0:
                    cw(s, s, j).wait_recv()
                    ccw(s, s, j).wait_recv()
                    p_cw[j] = p_cw[j] + comm_cw[
                        s, pl.ds(j * SUB, SUB), :].astype(jnp.float32)
                    p_ccw[j] = p_ccw[j] + comm_ccw[
                        s, pl.ds(j * SUB, SUB), :].astype(jnp.float32)
                if h >= 2:
                    cw(s, s, j).wait_send()
                    ccw(s, s, j).wait_send()
                comm_cw[s, pl.ds(j * SUB, SUB), :] = p_cw[j].astype(jnp.bfloat16)
                comm_ccw[s, pl.ds(j * SUB, SUB), :] = p_ccw[j].astype(jnp.bfloat16)
                cw(s, r, j).start()
                ccw(s, r, j).start()
            c_cw = lax.rem(me + N_DEV - h - 1, N_DEV)
            c_ccw = lax.rem(me + h + 1, N_DEV)
            p_cw = [partial_sub(c_cw, 0, j) for j in range(NSUB)]
            p_ccw = [partial_sub(c_ccw, HALF, j) for j in range(NSUB)]

        v_cw, v_ccw = [None] * NSUB, [None] * NSUB
        for j in range(NSUB):
            cw(1, 1, j).wait_recv()
            ccw(1, 1, j).wait_recv()
            v_cw[j] = p_cw[j] + comm_cw[
                1, pl.ds(j * SUB, SUB), :].astype(jnp.float32)
            v_ccw[j] = p_ccw[j] + comm_ccw[
                1, pl.ds(j * SUB, SUB), :].astype(jnp.float32)
            cw(1, 1, j).wait_send()
            ccw(1, 1, j).wait_send()
            comm_cw[1, pl.ds(j * SUB, SUB), :] = v_cw[j].astype(jnp.bfloat16)
            comm_ccw[1, pl.ds(j * SUB, SUB), :] = v_ccw[j].astype(jnp.bfloat16)
            cw(1, 0, j).start()
            ccw(1, 0, j).start()

        c_cw_fin = lax.rem(me + 1, N_DEV)
        c_ccw_fin = lax.rem(me + 3, N_DEV)
        store_half(0, jnp.concatenate(v_cw, axis=0), c_cw_fin, 0)
        store_half(1, jnp.concatenate(v_ccw, axis=0), c_ccw_fin, HALF)

        for g in range(1, N_DEV - 1):
            h = (N_DEV - 1) + g
            s = h % 2
            r = (h + 1) % 2
            for j in range(NSUB):
                cw(s, s, j).wait_recv()
                ccw(s, s, j).wait_recv()
                cw(s, s, j).wait_send()
                ccw(s, s, j).wait_send()
                cw(s, r, j).start()
                ccw(s, r, j).start()
            c_cw = lax.rem(me + 1 + N_DEV - g, N_DEV)
            c_ccw = lax.rem(me + 3 + g, N_DEV)
            store_half(0, comm_cw[s, :, :].astype(jnp.float32), c_cw, 0)
            store_half(1, comm_ccw[s, :, :].astype(jnp.float32), c_ccw, HALF)

        for j in range(NSUB):
            cw(0, 0, j).wait_recv()
            ccw(0, 0, j).wait_recv()
        c_last = lax.rem(me + 2, N_DEV)
        store_half(0, comm_cw[0, :, :].astype(jnp.float32), c_last, 0)
        store_half(1, comm_ccw[0, :, :].astype(jnp.float32), c_last, HALF)

        for j in range(NSUB):
            cw(0, 0, j).wait_send()
            ccw(0, 0, j).wait_send()
            cw(1, 1, j).wait_send()
            ccw(1, 1, j).wait_send()
        for d in range(2):
            pltpu.make_async_copy(
                stage_ref.at[d], stage_ref.at[d], stage_sems.at[d]
            ).wait()

    return pl.pallas_call(
        body,
        out_shape=jax.ShapeDtypeStruct((M, N), jnp.float32),
        in_specs=[
            pl.BlockSpec(memory_space=pltpu.MemorySpace.VMEM),
            pl.BlockSpec(memory_space=pltpu.MemorySpace.VMEM),
            pl.BlockSpec(memory_space=pltpu.MemorySpace.SMEM),
            pl.BlockSpec(memory_space=pltpu.MemorySpace.SMEM),
        ],
        out_specs=pl.BlockSpec(memory_space=pl.ANY),
        scratch_shapes=[
            pltpu.VMEM((CHUNK, N), jnp.bfloat16),
            pltpu.VMEM((2, HALF, N), jnp.bfloat16),
            pltpu.VMEM((2, HALF, N), jnp.bfloat16),
            pltpu.VMEM((2, HALF, N), jnp.float32),
            pltpu.SemaphoreType.DMA((2, NSUB)),
            pltpu.SemaphoreType.DMA((2, NSUB)),
            pltpu.SemaphoreType.DMA((2, NSUB)),
            pltpu.SemaphoreType.DMA((2, NSUB)),
            pltpu.SemaphoreType.DMA((2,)),
        ],
        compiler_params=pltpu.CompilerParams(
            collective_id=0,
            vmem_limit_bytes=100 * 1024 * 1024,
            skip_device_barrier=True,
        ),
    )(x, w_mat, scale_x, scale_w)
